# Initial kernel scaffold; baseline (speedup 1.0000x reference)
import jax
import jax.numpy as jnp
from jax.experimental import pallas as pl


def kernel(x, edge_index, batch, W1, b1, W2, b2, Wc, bc):
    raise NotImplementedError("write your pallas kernel here")


# speedup vs baseline: 25.1099x; 25.1099x over previous
import functools

import jax
import jax.numpy as jnp
from jax import lax
from jax.experimental import pallas as pl
from jax.experimental.pallas import tpu as pltpu
from jax.experimental.pallas import tpu_sc as plsc

N = 10000
E = 160000
D_IN = 256
D_H = 32
G = 64

NC = 2
NS = 16
NW = NC * NS
CHUNK = 128
CPW = 40
EW = CHUNK * CPW
E_PAD = EW * NW
RPT = 640
ACC_ROWS = RPT * NS
ROW_BLK = 1000
N_BLKS = N // ROW_BLK

_mesh = plsc.VectorSubcoreMesh(
    core_axis_name="c", subcore_axis_name="s", num_cores=NC, num_subcores=NS)

_sc_params = pltpu.CompilerParams(use_tc_tiling_on_sc=False)


@functools.partial(
    pl.kernel,
    out_type=jax.ShapeDtypeStruct((NC, ACC_ROWS, 16), jnp.float32),
    mesh=_mesh,
    compiler_params=_sc_params,
    scratch_types=[
        pltpu.VMEM_SHARED((ACC_ROWS, 16), jnp.float32),
        pltpu.VMEM((RPT, 16), jnp.float32),
        pltpu.VMEM((CPW, CHUNK), jnp.int32),
        pltpu.VMEM((CHUNK, 16), jnp.float32),
    ],
)
def _deg_kernel(dst2d, zeros_hbm, ones_hbm, out, acc, stage, dstv, onesv):
    cid = lax.axis_index("c")
    sid = lax.axis_index("s")
    wid = cid * NS + sid
    pltpu.sync_copy(zeros_hbm, stage)
    pltpu.sync_copy(stage, acc.at[pl.ds(sid * RPT, RPT)])
    pltpu.sync_copy(ones_hbm, onesv)
    pltpu.sync_copy(dst2d.at[pl.ds(wid * CPW, CPW)], dstv)
    plsc.subcore_barrier()

    def body(j, _):
        pltpu.sync_copy(onesv, acc.at[dstv.at[j]], add=True)
        return 0

    lax.fori_loop(0, CPW, body, 0)
    plsc.subcore_barrier()
    pltpu.sync_copy(acc.at[pl.ds(sid * RPT, RPT)], stage)
    pltpu.sync_copy(stage, out.at[cid].at[pl.ds(sid * RPT, RPT)])


@functools.partial(
    pl.kernel,
    out_type=jax.ShapeDtypeStruct((NC, ACC_ROWS, D_H), jnp.float32),
    mesh=_mesh,
    compiler_params=_sc_params,
    scratch_types=[
        pltpu.VMEM_SHARED((ACC_ROWS, D_H), jnp.float32),
        pltpu.VMEM_SHARED((ACC_ROWS, D_H), jnp.float32),
        pltpu.VMEM((RPT // 2, D_H), jnp.float32),
        pltpu.VMEM((CPW, CHUNK), jnp.int32),
        pltpu.VMEM((CPW, CHUNK), jnp.int32),
        pltpu.VMEM((CHUNK, D_H), jnp.float32),
    ],
)
def _mp_kernel(ht_pad, src2d, dst2d, zeros_hbm, out, acc, hts, stage, srcv,
               dstv, rows):
    cid = lax.axis_index("c")
    sid = lax.axis_index("s")
    wid = cid * NS + sid
    half = RPT // 2
    pltpu.sync_copy(zeros_hbm, stage)
    for hf in range(2):
        hr = pl.ds(sid * RPT + hf * half, half)
        pltpu.sync_copy(stage, acc.at[hr])
    for hf in range(2):
        hr = pl.ds(sid * RPT + hf * half, half)
        pltpu.sync_copy(ht_pad.at[hr], stage)
        pltpu.sync_copy(stage, hts.at[hr])
    pltpu.sync_copy(src2d.at[pl.ds(wid * CPW, CPW)], srcv)
    pltpu.sync_copy(dst2d.at[pl.ds(wid * CPW, CPW)], dstv)
    plsc.subcore_barrier()

    def body(j, _):
        pltpu.sync_copy(hts.at[srcv.at[j]], rows)
        pltpu.sync_copy(rows, acc.at[dstv.at[j]], add=True)
        return 0

    lax.fori_loop(0, CPW, body, 0)
    plsc.subcore_barrier()
    for hf in range(2):
        hr = pl.ds(sid * RPT + hf * half, half)
        pltpu.sync_copy(acc.at[hr], stage)
        pltpu.sync_copy(stage, out.at[cid].at[hr])


def _layer1_body(degp_ref, x_ref, w1_ref, ht_ref, dinv_ref):
    deg = degp_ref[0] + degp_ref[1]
    dinv = lax.rsqrt(deg[:, 0:1] + 1.0)
    h = jnp.dot(x_ref[...], w1_ref[...], preferred_element_type=jnp.float32)
    ht_ref[...] = h * dinv
    dinv_ref[...] = jnp.broadcast_to(dinv, (ROW_BLK, D_H))


def _layer2_body(p_ref, ht1_ref, dinv_ref, b1_ref, w2_ref, ht2_ref):
    s = p_ref[0] + p_ref[1] + ht1_ref[...]
    out1 = dinv_ref[...] * s + b1_ref[...]
    r = jnp.maximum(out1, 0.0)
    h2 = jnp.dot(r, w2_ref[...], preferred_element_type=jnp.float32)
    ht2_ref[...] = h2 * dinv_ref[...]


def _final_body(p_ref, ht2_ref, dinv_ref, b2_ref, batch_ref, wc_ref, bc_ref,
                out_ref):
    s = p_ref[0] + p_ref[1] + ht2_ref[...]
    out2 = dinv_ref[...] * s + b2_ref[...]
    gids = lax.broadcasted_iota(jnp.int32, (G, N), 0)
    onehot = (gids == jnp.broadcast_to(batch_ref[...], (G, N))
              ).astype(jnp.float32)
    pooled = jnp.dot(onehot, out2, preferred_element_type=jnp.float32)
    counts = jnp.sum(onehot, axis=1, keepdims=True)
    pooled = pooled / jnp.maximum(counts, 1.0)
    logits = (jnp.dot(pooled, wc_ref[...], preferred_element_type=jnp.float32)
              + bc_ref[...])
    m = jnp.max(logits, axis=1, keepdims=True)
    lse = jnp.log(jnp.sum(jnp.exp(logits - m), axis=1, keepdims=True)) + m
    out_ref[...] = logits - lse


def kernel(x, edge_index, batch, W1, b1, W2, b2, Wc, bc):
    f32 = jnp.float32
    src = edge_index[0]
    dst = edge_index[1]
    pad = E_PAD - E
    src2d = jnp.concatenate(
        [src, jnp.zeros((pad,), jnp.int32)]).reshape(E_PAD // CHUNK, CHUNK)
    dst2d = jnp.concatenate(
        [dst, jnp.full((pad,), N, jnp.int32)]).reshape(E_PAD // CHUNK, CHUNK)
    zeros_rows = jnp.zeros((RPT // 2, D_H), f32)
    zeros_rows16 = jnp.zeros((RPT, 16), f32)
    ones_rows = jnp.ones((CHUNK, 16), f32)

    degp = _deg_kernel(dst2d, zeros_rows16, ones_rows)

    ht1, dinv32 = pl.pallas_call(
        _layer1_body,
        grid=(N_BLKS,),
        in_specs=[
            pl.BlockSpec((NC, ROW_BLK, 16), lambda i: (0, i, 0)),
            pl.BlockSpec((ROW_BLK, D_IN), lambda i: (i, 0)),
            pl.BlockSpec((D_IN, D_H), lambda i: (0, 0)),
        ],
        out_specs=[
            pl.BlockSpec((ROW_BLK, D_H), lambda i: (i, 0)),
            pl.BlockSpec((ROW_BLK, D_H), lambda i: (i, 0)),
        ],
        out_shape=[
            jax.ShapeDtypeStruct((N, D_H), f32),
            jax.ShapeDtypeStruct((N, D_H), f32),
        ],
    )(degp, x, W1)

    row_pad = jnp.zeros((ACC_ROWS - N, D_H), f32)
    p1 = _mp_kernel(jnp.concatenate([ht1, row_pad]), src2d, dst2d, zeros_rows)

    ht2 = pl.pallas_call(
        _layer2_body,
        grid=(N_BLKS,),
        in_specs=[
            pl.BlockSpec((NC, ROW_BLK, D_H), lambda i: (0, i, 0)),
            pl.BlockSpec((ROW_BLK, D_H), lambda i: (i, 0)),
            pl.BlockSpec((ROW_BLK, D_H), lambda i: (i, 0)),
            pl.BlockSpec((1, D_H), lambda i: (0, 0)),
            pl.BlockSpec((D_H, D_H), lambda i: (0, 0)),
        ],
        out_specs=pl.BlockSpec((ROW_BLK, D_H), lambda i: (i, 0)),
        out_shape=jax.ShapeDtypeStruct((N, D_H), f32),
    )(p1, ht1, dinv32, b1.reshape(1, D_H), W2)

    p2 = _mp_kernel(jnp.concatenate([ht2, row_pad]), src2d, dst2d, zeros_rows)

    out = pl.pallas_call(
        _final_body,
        out_shape=jax.ShapeDtypeStruct((G, 2), f32),
    )(p2[:, :N, :], ht2, dinv32, b2.reshape(1, D_H),
      batch.reshape(1, N), Wc, bc.reshape(1, 2))
    return out


# speedup vs baseline: 26.3899x; 1.0510x over previous
import functools

import jax
import jax.numpy as jnp
from jax import lax
from jax.experimental import pallas as pl
from jax.experimental.pallas import tpu as pltpu
from jax.experimental.pallas import tpu_sc as plsc

N = 10000
E = 160000
D_IN = 256
D_H = 32
G = 64

NC = 2
NS = 16
NW = NC * NS
CHUNK = 256
CPW = 20
EW = CHUNK * CPW
E_PAD = EW * NW
DEG_CHUNK = 128
DEG_CPW = EW // DEG_CHUNK
RPT = 640
ACC_ROWS = RPT * NS
ROW_BLK = 1000
N_BLKS = N // ROW_BLK

_mesh = plsc.VectorSubcoreMesh(
    core_axis_name="c", subcore_axis_name="s", num_cores=NC, num_subcores=NS)

_sc_params = pltpu.CompilerParams(use_tc_tiling_on_sc=False)


@functools.partial(
    pl.kernel,
    out_type=jax.ShapeDtypeStruct((NC, ACC_ROWS, 16), jnp.float32),
    mesh=_mesh,
    compiler_params=_sc_params,
    scratch_types=[
        pltpu.VMEM_SHARED((ACC_ROWS, 16), jnp.float32),
        pltpu.VMEM((RPT, 16), jnp.float32),
        pltpu.VMEM((DEG_CPW, DEG_CHUNK), jnp.int32),
        pltpu.VMEM((DEG_CHUNK, 16), jnp.float32),
    ],
)
def _deg_kernel(dst2d, zeros_hbm, ones_hbm, out, acc, stage, dstv, onesv):
    cid = lax.axis_index("c")
    sid = lax.axis_index("s")
    wid = cid * NS + sid
    pltpu.sync_copy(zeros_hbm, stage)
    pltpu.sync_copy(stage, acc.at[pl.ds(sid * RPT, RPT)])
    pltpu.sync_copy(ones_hbm, onesv)
    pltpu.sync_copy(dst2d.at[pl.ds(wid * DEG_CPW, DEG_CPW)], dstv)
    plsc.subcore_barrier()

    def body(j, _):
        pltpu.sync_copy(onesv, acc.at[dstv.at[j]], add=True)
        return 0

    lax.fori_loop(0, DEG_CPW, body, 0)
    plsc.subcore_barrier()
    pltpu.sync_copy(acc.at[pl.ds(sid * RPT, RPT)], stage)
    pltpu.sync_copy(stage, out.at[cid].at[pl.ds(sid * RPT, RPT)])


@functools.partial(
    pl.kernel,
    out_type=jax.ShapeDtypeStruct((NC, ACC_ROWS, D_H), jnp.float32),
    mesh=_mesh,
    compiler_params=_sc_params,
    scratch_types=[
        pltpu.VMEM_SHARED((ACC_ROWS, D_H), jnp.float32),
        pltpu.VMEM((RPT // 2, D_H), jnp.float32),
        pltpu.VMEM((EW,), jnp.int32),
        pltpu.VMEM((EW,), jnp.int32),
        pltpu.VMEM((CHUNK, D_H), jnp.float32),
        pltpu.VMEM((CHUNK, D_H), jnp.float32),
        pltpu.SemaphoreType.DMA,
        pltpu.SemaphoreType.DMA,
    ],
)
def _mp_kernel(ht, src_flat, dst_flat, zeros_hbm, out, acc, stage, srcv,
               dstv, rows0, rows1, sem0, sem1):
    cid = lax.axis_index("c")
    sid = lax.axis_index("s")
    wid = cid * NS + sid
    half = RPT // 2
    pltpu.sync_copy(zeros_hbm, stage)
    for hf in range(2):
        hr = pl.ds(sid * RPT + hf * half, half)
        pltpu.sync_copy(stage, acc.at[hr])
    base = wid * EW
    pltpu.sync_copy(src_flat.at[pl.ds(base, EW)], srcv)
    pltpu.sync_copy(dst_flat.at[pl.ds(base, EW)], dstv)
    plsc.subcore_barrier()

    rows = (rows0, rows1)
    sems = (sem0, sem1)
    pltpu.async_copy(ht.at[srcv.at[pl.ds(0, CHUNK)]], rows0, sem0)

    def body(j2, _):
        for p in range(2):
            j = j2 * 2 + p
            pltpu.make_async_copy(ht.at[srcv.at[pl.ds(0, CHUNK)]],
                                  rows[p], sems[p]).wait()

            @pl.when(j + 1 < CPW)
            def _():
                nxt = pl.ds((j + 1) * CHUNK, CHUNK)
                pltpu.async_copy(ht.at[srcv.at[nxt]], rows[1 - p],
                                 sems[1 - p])

            cur = pl.ds(j * CHUNK, CHUNK)
            pltpu.sync_copy(rows[p], acc.at[dstv.at[cur]], add=True)
        return 0

    lax.fori_loop(0, CPW // 2, body, 0)
    plsc.subcore_barrier()
    for hf in range(2):
        hr = pl.ds(sid * RPT + hf * half, half)
        pltpu.sync_copy(acc.at[hr], stage)
        pltpu.sync_copy(stage, out.at[cid].at[hr])


def _layer1_body(degp_ref, x_ref, w1_ref, ht_ref, dinv_ref):
    deg = degp_ref[0] + degp_ref[1]
    dinv = lax.rsqrt(deg[:, 0:1] + 1.0)
    h = jnp.dot(x_ref[...], w1_ref[...], preferred_element_type=jnp.float32)
    ht_ref[...] = h * dinv
    dinv_ref[...] = jnp.broadcast_to(dinv, (ROW_BLK, D_H))


def _layer2_body(p_ref, ht1_ref, dinv_ref, b1_ref, w2_ref, ht2_ref):
    s = p_ref[0] + p_ref[1] + ht1_ref[...]
    out1 = dinv_ref[...] * s + b1_ref[...]
    r = jnp.maximum(out1, 0.0)
    h2 = jnp.dot(r, w2_ref[...], preferred_element_type=jnp.float32)
    ht2_ref[...] = h2 * dinv_ref[...]


def _final_body(p_ref, ht2_ref, dinv_ref, b2_ref, batch_ref, wc_ref, bc_ref,
                out_ref):
    s = p_ref[0] + p_ref[1] + ht2_ref[...]
    out2 = dinv_ref[...] * s + b2_ref[...]
    gids = lax.broadcasted_iota(jnp.int32, (G, N), 0)
    onehot = (gids == jnp.broadcast_to(batch_ref[...], (G, N))
              ).astype(jnp.float32)
    pooled = jnp.dot(onehot, out2, preferred_element_type=jnp.float32)
    counts = jnp.sum(onehot, axis=1, keepdims=True)
    pooled = pooled / jnp.maximum(counts, 1.0)
    logits = (jnp.dot(pooled, wc_ref[...], preferred_element_type=jnp.float32)
              + bc_ref[...])
    m = jnp.max(logits, axis=1, keepdims=True)
    lse = jnp.log(jnp.sum(jnp.exp(logits - m), axis=1, keepdims=True)) + m
    out_ref[...] = logits - lse


def kernel(x, edge_index, batch, W1, b1, W2, b2, Wc, bc):
    f32 = jnp.float32
    src = edge_index[0]
    dst = edge_index[1]
    pad = E_PAD - E
    pad_ar = jnp.arange(pad, dtype=jnp.int32)
    src_flat = jnp.concatenate([src, pad_ar % N])
    dst_pad = N + (pad_ar % (ACC_ROWS - N))
    dst_flat = jnp.concatenate([dst, dst_pad])
    dst2d = dst_flat.reshape(E_PAD // DEG_CHUNK, DEG_CHUNK)
    zeros_rows = jnp.zeros((RPT // 2, D_H), f32)
    zeros_rows16 = jnp.zeros((RPT, 16), f32)
    ones_rows = jnp.ones((DEG_CHUNK, 16), f32)

    degp = _deg_kernel(dst2d, zeros_rows16, ones_rows)

    ht1, dinv32 = pl.pallas_call(
        _layer1_body,
        grid=(N_BLKS,),
        in_specs=[
            pl.BlockSpec((NC, ROW_BLK, 16), lambda i: (0, i, 0)),
            pl.BlockSpec((ROW_BLK, D_IN), lambda i: (i, 0)),
            pl.BlockSpec((D_IN, D_H), lambda i: (0, 0)),
        ],
        out_specs=[
            pl.BlockSpec((ROW_BLK, D_H), lambda i: (i, 0)),
            pl.BlockSpec((ROW_BLK, D_H), lambda i: (i, 0)),
        ],
        out_shape=[
            jax.ShapeDtypeStruct((N, D_H), f32),
            jax.ShapeDtypeStruct((N, D_H), f32),
        ],
    )(degp, x, W1)

    p1 = _mp_kernel(ht1, src_flat, dst_flat, zeros_rows)

    ht2 = pl.pallas_call(
        _layer2_body,
        grid=(N_BLKS,),
        in_specs=[
            pl.BlockSpec((NC, ROW_BLK, D_H), lambda i: (0, i, 0)),
            pl.BlockSpec((ROW_BLK, D_H), lambda i: (i, 0)),
            pl.BlockSpec((ROW_BLK, D_H), lambda i: (i, 0)),
            pl.BlockSpec((1, D_H), lambda i: (0, 0)),
            pl.BlockSpec((D_H, D_H), lambda i: (0, 0)),
        ],
        out_specs=pl.BlockSpec((ROW_BLK, D_H), lambda i: (i, 0)),
        out_shape=jax.ShapeDtypeStruct((N, D_H), f32),
    )(p1, ht1, dinv32, b1.reshape(1, D_H), W2)

    p2 = _mp_kernel(ht2, src_flat, dst_flat, zeros_rows)

    out = pl.pallas_call(
        _final_body,
        out_shape=jax.ShapeDtypeStruct((G, 2), f32),
    )(p2[:, :N, :], ht2, dinv32, b2.reshape(1, D_H),
      batch.reshape(1, N), Wc, bc.reshape(1, 2))
    return out


# speedup vs baseline: 28.0425x; 1.0626x over previous
import functools

import jax
import jax.numpy as jnp
from jax import lax
from jax.experimental import pallas as pl
from jax.experimental.pallas import tpu as pltpu
from jax.experimental.pallas import tpu_sc as plsc

N = 10000
E = 160000
D_IN = 256
D_H = 32
G = 64

NC = 2
NS = 16
NW = NC * NS
CHUNK = 200
CPW = 25
EW = CHUNK * CPW
RPT = 640
ACC_ROWS = RPT * NS
ROW_BLK = 1000
N_BLKS = N // ROW_BLK

_mesh = plsc.VectorSubcoreMesh(
    core_axis_name="c", subcore_axis_name="s", num_cores=NC, num_subcores=NS)

_sc_params = pltpu.CompilerParams(use_tc_tiling_on_sc=False)


@functools.partial(
    pl.kernel,
    out_type=jax.ShapeDtypeStruct((NC, ACC_ROWS, 16), jnp.float32),
    mesh=_mesh,
    compiler_params=_sc_params,
    scratch_types=[
        pltpu.VMEM_SHARED((ACC_ROWS, 16), jnp.float32),
        pltpu.VMEM((RPT, 16), jnp.float32),
        pltpu.VMEM((EW,), jnp.int32),
        pltpu.VMEM((CHUNK, 16), jnp.float32),
    ],
)
def _deg_kernel(edge_index, zeros_hbm, ones_hbm, out, acc, stage, dstv,
                onesv):
    cid = lax.axis_index("c")
    sid = lax.axis_index("s")
    wid = cid * NS + sid
    pltpu.sync_copy(zeros_hbm, stage)
    pltpu.sync_copy(stage, acc.at[pl.ds(sid * RPT, RPT)])
    pltpu.sync_copy(ones_hbm, onesv)
    pltpu.sync_copy(edge_index.at[1].at[pl.ds(wid * EW, EW)], dstv)
    plsc.subcore_barrier()

    def body(j, _):
        pltpu.sync_copy(onesv, acc.at[dstv.at[pl.ds(j * CHUNK, CHUNK)]],
                        add=True)
        return 0

    lax.fori_loop(0, CPW, body, 0)
    plsc.subcore_barrier()
    pltpu.sync_copy(acc.at[pl.ds(sid * RPT, RPT)], stage)
    pltpu.sync_copy(stage, out.at[cid].at[pl.ds(sid * RPT, RPT)])


@functools.partial(
    pl.kernel,
    out_type=jax.ShapeDtypeStruct((NC, ACC_ROWS, D_H), jnp.float32),
    mesh=_mesh,
    compiler_params=_sc_params,
    scratch_types=[
        pltpu.VMEM_SHARED((ACC_ROWS, D_H), jnp.float32),
        pltpu.VMEM((RPT // 2, D_H), jnp.float32),
        pltpu.VMEM((EW,), jnp.int32),
        pltpu.VMEM((EW,), jnp.int32),
        pltpu.VMEM((CHUNK, D_H), jnp.float32),
        pltpu.VMEM((CHUNK, D_H), jnp.float32),
        pltpu.SemaphoreType.DMA,
        pltpu.SemaphoreType.DMA,
    ],
)
def _mp_kernel(ht, edge_index, zeros_hbm, out, acc, stage, srcv,
               dstv, rows0, rows1, sem0, sem1):
    cid = lax.axis_index("c")
    sid = lax.axis_index("s")
    wid = cid * NS + sid
    half = RPT // 2
    pltpu.sync_copy(zeros_hbm, stage)
    for hf in range(2):
        hr = pl.ds(sid * RPT + hf * half, half)
        pltpu.sync_copy(stage, acc.at[hr])
    base = wid * EW
    pltpu.sync_copy(edge_index.at[0].at[pl.ds(base, EW)], srcv)
    pltpu.sync_copy(edge_index.at[1].at[pl.ds(base, EW)], dstv)
    plsc.subcore_barrier()

    rows = (rows0, rows1)
    sems = (sem0, sem1)
    pltpu.async_copy(ht.at[srcv.at[pl.ds(0, CHUNK)]], rows0, sem0)

    def body(j2, _):
        for p in range(2):
            j = j2 * 2 + p
            pltpu.make_async_copy(ht.at[srcv.at[pl.ds(0, CHUNK)]],
                                  rows[p], sems[p]).wait()

            @pl.when(j + 1 < CPW)
            def _():
                nxt = pl.ds((j + 1) * CHUNK, CHUNK)
                pltpu.async_copy(ht.at[srcv.at[nxt]], rows[1 - p],
                                 sems[1 - p])

            cur = pl.ds(j * CHUNK, CHUNK)
            pltpu.sync_copy(rows[p], acc.at[dstv.at[cur]], add=True)
        return 0

    lax.fori_loop(0, (CPW - 1) // 2, body, 0)
    pltpu.make_async_copy(ht.at[srcv.at[pl.ds(0, CHUNK)]], rows0,
                          sem0).wait()
    pltpu.sync_copy(rows0, acc.at[dstv.at[pl.ds((CPW - 1) * CHUNK, CHUNK)]],
                    add=True)
    plsc.subcore_barrier()
    for hf in range(2):
        hr = pl.ds(sid * RPT + hf * half, half)
        pltpu.sync_copy(acc.at[hr], stage)
        pltpu.sync_copy(stage, out.at[cid].at[hr])


def _layer1_body(degp_ref, x_ref, w1_ref, ht_ref, dinv_ref):
    deg = degp_ref[0] + degp_ref[1]
    dinv = lax.rsqrt(deg[:, 0:1] + 1.0)
    h = jnp.dot(x_ref[...], w1_ref[...], preferred_element_type=jnp.float32)
    ht_ref[...] = h * dinv
    dinv_ref[...] = jnp.broadcast_to(dinv, (ROW_BLK, D_H))


def _layer2_body(p_ref, ht1_ref, dinv_ref, b1_ref, w2_ref, ht2_ref):
    s = p_ref[0] + p_ref[1] + ht1_ref[...]
    out1 = dinv_ref[...] * s + b1_ref[...]
    r = jnp.maximum(out1, 0.0)
    h2 = jnp.dot(r, w2_ref[...], preferred_element_type=jnp.float32)
    ht2_ref[...] = h2 * dinv_ref[...]


def _final_body(p_ref, ht2_ref, dinv_ref, b2_ref, batch_ref, wc_ref, bc_ref,
                out_ref):
    s = p_ref[0][:N] + p_ref[1][:N] + ht2_ref[...]
    out2 = dinv_ref[...] * s + b2_ref[...]
    gids = lax.broadcasted_iota(jnp.int32, (G, N), 0)
    onehot = (gids == jnp.broadcast_to(batch_ref[...], (G, N))
              ).astype(jnp.float32)
    pooled = jnp.dot(onehot, out2, preferred_element_type=jnp.float32)
    counts = jnp.sum(onehot, axis=1, keepdims=True)
    pooled = pooled / jnp.maximum(counts, 1.0)
    logits = (jnp.dot(pooled, wc_ref[...], preferred_element_type=jnp.float32)
              + bc_ref[...])
    m = jnp.max(logits, axis=1, keepdims=True)
    lse = jnp.log(jnp.sum(jnp.exp(logits - m), axis=1, keepdims=True)) + m
    out_ref[...] = logits - lse


def kernel(x, edge_index, batch, W1, b1, W2, b2, Wc, bc):
    f32 = jnp.float32
    zeros_rows = jnp.zeros((RPT // 2, D_H), f32)
    zeros_rows16 = jnp.zeros((RPT, 16), f32)
    ones_rows = jnp.ones((CHUNK, 16), f32)

    degp = _deg_kernel(edge_index, zeros_rows16, ones_rows)

    ht1, dinv32 = pl.pallas_call(
        _layer1_body,
        grid=(N_BLKS,),
        in_specs=[
            pl.BlockSpec((NC, ROW_BLK, 16), lambda i: (0, i, 0)),
            pl.BlockSpec((ROW_BLK, D_IN), lambda i: (i, 0)),
            pl.BlockSpec((D_IN, D_H), lambda i: (0, 0)),
        ],
        out_specs=[
            pl.BlockSpec((ROW_BLK, D_H), lambda i: (i, 0)),
            pl.BlockSpec((ROW_BLK, D_H), lambda i: (i, 0)),
        ],
        out_shape=[
            jax.ShapeDtypeStruct((N, D_H), f32),
            jax.ShapeDtypeStruct((N, D_H), f32),
        ],
    )(degp, x, W1)

    p1 = _mp_kernel(ht1, edge_index, zeros_rows)

    ht2 = pl.pallas_call(
        _layer2_body,
        grid=(N_BLKS,),
        in_specs=[
            pl.BlockSpec((NC, ROW_BLK, D_H), lambda i: (0, i, 0)),
            pl.BlockSpec((ROW_BLK, D_H), lambda i: (i, 0)),
            pl.BlockSpec((ROW_BLK, D_H), lambda i: (i, 0)),
            pl.BlockSpec((1, D_H), lambda i: (0, 0)),
            pl.BlockSpec((D_H, D_H), lambda i: (0, 0)),
        ],
        out_specs=pl.BlockSpec((ROW_BLK, D_H), lambda i: (i, 0)),
        out_shape=jax.ShapeDtypeStruct((N, D_H), f32),
    )(p1, ht1, dinv32, b1.reshape(1, D_H), W2)

    p2 = _mp_kernel(ht2, edge_index, zeros_rows)

    out = pl.pallas_call(
        _final_body,
        out_shape=jax.ShapeDtypeStruct((G, 2), f32),
    )(p2, ht2, dinv32, b2.reshape(1, D_H),
      batch.reshape(1, N), Wc, bc.reshape(1, 2))
    return out


# speedup vs baseline: 28.8405x; 1.0285x over previous
import functools

import jax
import jax.numpy as jnp
from jax import lax
from jax.experimental import pallas as pl
from jax.experimental.pallas import tpu as pltpu
from jax.experimental.pallas import tpu_sc as plsc

N = 10000
E = 160000
D_IN = 256
D_H = 32
G = 64

NC = 2
NS = 16
NW = NC * NS
CHUNK = 200
CPW = 25
EW = CHUNK * CPW
RPT = 640
ACC_ROWS = RPT * NS
ROW_BLK = 2000
N_BLKS = N // ROW_BLK

_mesh = plsc.VectorSubcoreMesh(
    core_axis_name="c", subcore_axis_name="s", num_cores=NC, num_subcores=NS)

_sc_params = pltpu.CompilerParams(use_tc_tiling_on_sc=False)


@functools.partial(
    pl.kernel,
    out_type=jax.ShapeDtypeStruct((NC, ACC_ROWS, 16), jnp.float32),
    mesh=_mesh,
    compiler_params=_sc_params,
    scratch_types=[
        pltpu.VMEM_SHARED((ACC_ROWS, 16), jnp.float32),
        pltpu.VMEM((RPT, 16), jnp.float32),
        pltpu.VMEM((EW,), jnp.int32),
        pltpu.VMEM((CHUNK, 16), jnp.float32),
    ],
)
def _deg_kernel(edge_index, zeros_hbm, ones_hbm, out, acc, stage, dstv,
                onesv):
    cid = lax.axis_index("c")
    sid = lax.axis_index("s")
    wid = cid * NS + sid
    pltpu.sync_copy(zeros_hbm, stage)
    pltpu.sync_copy(stage, acc.at[pl.ds(sid * RPT, RPT)])
    pltpu.sync_copy(ones_hbm, onesv)
    pltpu.sync_copy(edge_index.at[1].at[pl.ds(wid * EW, EW)], dstv)
    plsc.subcore_barrier()

    def body(j, _):
        pltpu.sync_copy(onesv, acc.at[dstv.at[pl.ds(j * CHUNK, CHUNK)]],
                        add=True)
        return 0

    lax.fori_loop(0, CPW, body, 0)
    plsc.subcore_barrier()
    pltpu.sync_copy(acc.at[pl.ds(sid * RPT, RPT)], stage)
    pltpu.sync_copy(stage, out.at[cid].at[pl.ds(sid * RPT, RPT)])


@functools.partial(
    pl.kernel,
    out_type=jax.ShapeDtypeStruct((NC, ACC_ROWS, D_H), jnp.float32),
    mesh=_mesh,
    compiler_params=_sc_params,
    scratch_types=[
        pltpu.VMEM_SHARED((ACC_ROWS, D_H), jnp.float32),
        pltpu.VMEM((RPT // 2, D_H), jnp.float32),
        pltpu.VMEM((EW,), jnp.int32),
        pltpu.VMEM((EW,), jnp.int32),
        pltpu.VMEM((CHUNK, D_H), jnp.float32),
        pltpu.VMEM((CHUNK, D_H), jnp.float32),
        pltpu.SemaphoreType.DMA,
        pltpu.SemaphoreType.DMA,
    ],
)
def _mp_kernel(ht, edge_index, zeros_hbm, out, acc, stage, srcv,
               dstv, rows0, rows1, sem0, sem1):
    cid = lax.axis_index("c")
    sid = lax.axis_index("s")
    wid = cid * NS + sid
    half = RPT // 2
    pltpu.sync_copy(zeros_hbm, stage)
    for hf in range(2):
        hr = pl.ds(sid * RPT + hf * half, half)
        pltpu.sync_copy(stage, acc.at[hr])
    base = wid * EW
    pltpu.sync_copy(edge_index.at[0].at[pl.ds(base, EW)], srcv)
    pltpu.sync_copy(edge_index.at[1].at[pl.ds(base, EW)], dstv)
    plsc.subcore_barrier()

    rows = (rows0, rows1)
    sems = (sem0, sem1)
    pltpu.async_copy(ht.at[srcv.at[pl.ds(0, CHUNK)]], rows0, sem0)

    def body(j2, _):
        for p in range(2):
            j = j2 * 2 + p
            pltpu.make_async_copy(ht.at[srcv.at[pl.ds(0, CHUNK)]],
                                  rows[p], sems[p]).wait()

            @pl.when(j + 1 < CPW)
            def _():
                nxt = pl.ds((j + 1) * CHUNK, CHUNK)
                pltpu.async_copy(ht.at[srcv.at[nxt]], rows[1 - p],
                                 sems[1 - p])

            cur = pl.ds(j * CHUNK, CHUNK)
            pltpu.sync_copy(rows[p], acc.at[dstv.at[cur]], add=True)
        return 0

    lax.fori_loop(0, (CPW - 1) // 2, body, 0)
    pltpu.make_async_copy(ht.at[srcv.at[pl.ds(0, CHUNK)]], rows0,
                          sem0).wait()
    pltpu.sync_copy(rows0, acc.at[dstv.at[pl.ds((CPW - 1) * CHUNK, CHUNK)]],
                    add=True)
    plsc.subcore_barrier()
    for hf in range(2):
        hr = pl.ds(sid * RPT + hf * half, half)
        pltpu.sync_copy(acc.at[hr], stage)
        pltpu.sync_copy(stage, out.at[cid].at[hr])


def _layer1_body(degp_ref, x_ref, w1_ref, ht_ref, dinv_ref):
    deg = degp_ref[0] + degp_ref[1]
    dinv = lax.rsqrt(deg[:, 0:1] + 1.0)
    h = jnp.dot(x_ref[...], w1_ref[...], preferred_element_type=jnp.float32)
    ht_ref[...] = h * dinv
    dinv_ref[...] = jnp.broadcast_to(dinv, (ROW_BLK, D_H))


def _layer2_body(p_ref, ht1_ref, dinv_ref, b1_ref, w2_ref, ht2_ref):
    s = p_ref[0] + p_ref[1] + ht1_ref[...]
    out1 = dinv_ref[...] * s + b1_ref[...]
    r = jnp.maximum(out1, 0.0)
    h2 = jnp.dot(r, w2_ref[...], preferred_element_type=jnp.float32)
    ht2_ref[...] = h2 * dinv_ref[...]


def _final_body(p_ref, ht2_ref, dinv_ref, b2_ref, batch_ref, wc_ref, bc_ref,
                out_ref):
    s = p_ref[0][:N] + p_ref[1][:N] + ht2_ref[...]
    out2 = dinv_ref[...] * s + b2_ref[...]
    gids = lax.broadcasted_iota(jnp.int32, (G, N), 0)
    onehot = (gids == jnp.broadcast_to(batch_ref[...], (G, N))
              ).astype(jnp.float32)
    pooled = jnp.dot(onehot, out2, preferred_element_type=jnp.float32)
    counts = jnp.sum(onehot, axis=1, keepdims=True)
    pooled = pooled / jnp.maximum(counts, 1.0)
    logits = (jnp.dot(pooled, wc_ref[...], preferred_element_type=jnp.float32)
              + bc_ref[...])
    m = jnp.max(logits, axis=1, keepdims=True)
    lse = jnp.log(jnp.sum(jnp.exp(logits - m), axis=1, keepdims=True)) + m
    out_ref[...] = logits - lse


def kernel(x, edge_index, batch, W1, b1, W2, b2, Wc, bc):
    f32 = jnp.float32
    zeros_rows = jnp.zeros((RPT // 2, D_H), f32)
    zeros_rows16 = jnp.zeros((RPT, 16), f32)
    ones_rows = jnp.ones((CHUNK, 16), f32)

    degp = _deg_kernel(edge_index, zeros_rows16, ones_rows)

    ht1, dinv32 = pl.pallas_call(
        _layer1_body,
        grid=(N_BLKS,),
        in_specs=[
            pl.BlockSpec((NC, ROW_BLK, 16), lambda i: (0, i, 0)),
            pl.BlockSpec((ROW_BLK, D_IN), lambda i: (i, 0)),
            pl.BlockSpec((D_IN, D_H), lambda i: (0, 0)),
        ],
        out_specs=[
            pl.BlockSpec((ROW_BLK, D_H), lambda i: (i, 0)),
            pl.BlockSpec((ROW_BLK, D_H), lambda i: (i, 0)),
        ],
        out_shape=[
            jax.ShapeDtypeStruct((N, D_H), f32),
            jax.ShapeDtypeStruct((N, D_H), f32),
        ],
    )(degp, x, W1)

    p1 = _mp_kernel(ht1, edge_index, zeros_rows)

    ht2 = pl.pallas_call(
        _layer2_body,
        grid=(N_BLKS,),
        in_specs=[
            pl.BlockSpec((NC, ROW_BLK, D_H), lambda i: (0, i, 0)),
            pl.BlockSpec((ROW_BLK, D_H), lambda i: (i, 0)),
            pl.BlockSpec((ROW_BLK, D_H), lambda i: (i, 0)),
            pl.BlockSpec((1, D_H), lambda i: (0, 0)),
            pl.BlockSpec((D_H, D_H), lambda i: (0, 0)),
        ],
        out_specs=pl.BlockSpec((ROW_BLK, D_H), lambda i: (i, 0)),
        out_shape=jax.ShapeDtypeStruct((N, D_H), f32),
    )(p1, ht1, dinv32, b1.reshape(1, D_H), W2)

    p2 = _mp_kernel(ht2, edge_index, zeros_rows)

    out = pl.pallas_call(
        _final_body,
        out_shape=jax.ShapeDtypeStruct((G, 2), f32),
    )(p2, ht2, dinv32, b2.reshape(1, D_H),
      batch.reshape(1, N), Wc, bc.reshape(1, 2))
    return out


# speedup vs baseline: 38.7938x; 1.3451x over previous
import functools

import jax
import jax.numpy as jnp
from jax import lax
from jax.experimental import pallas as pl
from jax.experimental.pallas import tpu as pltpu
from jax.experimental.pallas import tpu_sc as plsc

N = 10000
E = 160000
D_IN = 256
D_H = 32
G = 64

NC = 2
NS = 16
NW = NC * NS
CHUNK = 200
CPW = 25
EW = CHUNK * CPW
RPT = 640
ACC_ROWS = RPT * NS
ROW_BLK = 2000
N_BLKS = N // ROW_BLK

_mesh = plsc.VectorSubcoreMesh(
    core_axis_name="c", subcore_axis_name="s", num_cores=NC, num_subcores=NS)

_sc_params = pltpu.CompilerParams(use_tc_tiling_on_sc=False)


@functools.partial(
    pl.kernel,
    out_type=jax.ShapeDtypeStruct((NC, ACC_ROWS, D_H), jnp.float32),
    mesh=_mesh,
    compiler_params=_sc_params,
    scratch_types=[
        pltpu.VMEM_SHARED((ACC_ROWS, D_H), jnp.float32),
        pltpu.VMEM((RPT // 2, D_H), jnp.float32),
        pltpu.VMEM((EW,), jnp.int32),
        pltpu.VMEM((CHUNK, D_H), jnp.float32),
    ],
)
def _deg_kernel(edge_index, zeros_hbm, ones_hbm, out, acc, stage, dstv,
                onesv):
    cid = lax.axis_index("c")
    sid = lax.axis_index("s")
    wid = cid * NS + sid
    half = RPT // 2
    pltpu.sync_copy(zeros_hbm, stage)
    for hf in range(2):
        pltpu.sync_copy(stage, acc.at[pl.ds(sid * RPT + hf * half, half)])
    pltpu.sync_copy(ones_hbm, onesv)
    pltpu.sync_copy(edge_index.at[1].at[pl.ds(wid * EW, EW)], dstv)
    plsc.subcore_barrier()

    def body(j, _):
        pltpu.sync_copy(onesv, acc.at[dstv.at[pl.ds(j * CHUNK, CHUNK)]],
                        add=True)
        return 0

    lax.fori_loop(0, CPW, body, 0)
    plsc.subcore_barrier()
    for hf in range(2):
        hr = pl.ds(sid * RPT + hf * half, half)
        pltpu.sync_copy(acc.at[hr], stage)
        pltpu.sync_copy(stage, out.at[cid].at[hr])


@functools.partial(
    pl.kernel,
    out_type=jax.ShapeDtypeStruct((NC, ACC_ROWS, D_H), jnp.float32),
    mesh=_mesh,
    compiler_params=_sc_params,
    scratch_types=[
        pltpu.VMEM_SHARED((ACC_ROWS, D_H), jnp.float32),
        pltpu.VMEM((RPT // 2, D_H), jnp.float32),
        pltpu.VMEM((EW,), jnp.int32),
        pltpu.VMEM((EW,), jnp.int32),
        pltpu.VMEM((CHUNK, D_H), jnp.float32),
        pltpu.VMEM((CHUNK, D_H), jnp.float32),
        pltpu.SemaphoreType.DMA,
        pltpu.SemaphoreType.DMA,
    ],
)
def _mp_kernel(ht, edge_index, zeros_hbm, out, acc, stage, srcv,
               dstv, rows0, rows1, sem0, sem1):
    cid = lax.axis_index("c")
    sid = lax.axis_index("s")
    wid = cid * NS + sid
    half = RPT // 2
    pltpu.sync_copy(zeros_hbm, stage)
    for hf in range(2):
        hr = pl.ds(sid * RPT + hf * half, half)
        pltpu.sync_copy(stage, acc.at[hr])
    base = wid * EW
    pltpu.sync_copy(edge_index.at[0].at[pl.ds(base, EW)], srcv)
    pltpu.sync_copy(edge_index.at[1].at[pl.ds(base, EW)], dstv)
    plsc.subcore_barrier()

    rows = (rows0, rows1)
    sems = (sem0, sem1)
    pltpu.async_copy(ht.at[srcv.at[pl.ds(0, CHUNK)]], rows0, sem0)

    def body(j2, _):
        for p in range(2):
            j = j2 * 2 + p
            pltpu.make_async_copy(ht.at[srcv.at[pl.ds(0, CHUNK)]],
                                  rows[p], sems[p]).wait()

            @pl.when(j + 1 < CPW)
            def _():
                nxt = pl.ds((j + 1) * CHUNK, CHUNK)
                pltpu.async_copy(ht.at[srcv.at[nxt]], rows[1 - p],
                                 sems[1 - p])

            cur = pl.ds(j * CHUNK, CHUNK)
            pltpu.sync_copy(rows[p], acc.at[dstv.at[cur]], add=True)
        return 0

    lax.fori_loop(0, (CPW - 1) // 2, body, 0)
    pltpu.make_async_copy(ht.at[srcv.at[pl.ds(0, CHUNK)]], rows0,
                          sem0).wait()
    pltpu.sync_copy(rows0, acc.at[dstv.at[pl.ds((CPW - 1) * CHUNK, CHUNK)]],
                    add=True)
    plsc.subcore_barrier()
    for hf in range(2):
        hr = pl.ds(sid * RPT + hf * half, half)
        pltpu.sync_copy(acc.at[hr], stage)
        pltpu.sync_copy(stage, out.at[cid].at[hr])


P_V = ACC_ROWS * D_H // 128
HT_V = N * D_H // 128
NB = N // 4


def _blockdiag4(w, d):
    cols = []
    for k in range(4):
        parts = []
        if k > 0:
            parts.append(jnp.zeros((k * d, D_H), jnp.float32))
        parts.append(w)
        if k < 3:
            parts.append(jnp.zeros(((3 - k) * d, D_H), jnp.float32))
        cols.append(jnp.concatenate(parts, axis=0))
    return jnp.concatenate(cols, axis=1)


def _tile4(v):
    return jnp.concatenate([v, v, v, v], axis=1)


def _layer1_body(x4_ref, w1_ref, h1_ref):
    w1b = _blockdiag4(w1_ref[...], D_IN)
    h1_ref[...] = jnp.dot(x4_ref[...], w1b,
                          preferred_element_type=jnp.float32)


def _scale1_body(degp_ref, h1_ref, ht_ref, dinv_ref):
    deg = degp_ref[0] + degp_ref[1] + 1.0
    dinv = lax.rsqrt(deg)[:HT_V]
    dinv_ref[...] = dinv
    ht_ref[...] = h1_ref[...] * dinv


def _layer2_body(p_ref, ht1_ref, dinv_ref, b1_ref, w2_ref, ht2_ref):
    p = p_ref[0][:HT_V] + p_ref[1][:HT_V]
    dinv = dinv_ref[...]
    out1 = dinv * (p + ht1_ref[...]) + _tile4(b1_ref[...])
    r = jnp.maximum(out1, 0.0)
    w2b = _blockdiag4(w2_ref[...], D_H)
    h2 = jnp.dot(r, w2b, preferred_element_type=jnp.float32)
    ht2_ref[...] = h2 * dinv


def _final_body(p_ref, ht2_ref, dinv_ref, b2_ref, batcht_ref, wc_ref, bc_ref,
                out_ref):
    p = p_ref[0][:HT_V] + p_ref[1][:HT_V]
    out2 = dinv_ref[...] * (p + ht2_ref[...]) + _tile4(b2_ref[...])
    pooled = jnp.zeros((G, D_H), jnp.float32)
    counts = jnp.zeros((G, 1), jnp.float32)
    gids = lax.broadcasted_iota(jnp.int32, (G, NB), 0)
    for k in range(4):
        bk = batcht_ref[k:k + 1, :]
        ohk = (gids == jnp.broadcast_to(bk, (G, NB))).astype(jnp.float32)
        pk = jnp.dot(ohk, out2, preferred_element_type=jnp.float32)
        pooled = pooled + pk[:, k * D_H:(k + 1) * D_H]
        counts = counts + jnp.sum(ohk, axis=1, keepdims=True)
    pooled = pooled / jnp.maximum(counts, 1.0)
    logits = (jnp.dot(pooled, wc_ref[...], preferred_element_type=jnp.float32)
              + bc_ref[...])
    m = jnp.max(logits, axis=1, keepdims=True)
    lse = jnp.log(jnp.sum(jnp.exp(logits - m), axis=1, keepdims=True)) + m
    out_ref[...] = logits - lse


def kernel(x, edge_index, batch, W1, b1, W2, b2, Wc, bc):
    f32 = jnp.float32
    zeros_rows = jnp.zeros((RPT // 2, D_H), f32)
    ones_rows = jnp.ones((CHUNK, D_H), f32)

    degp = _deg_kernel(edge_index, zeros_rows, ones_rows)

    h1v = pl.pallas_call(
        _layer1_body,
        out_shape=jax.ShapeDtypeStruct((HT_V, 128), f32),
    )(x.reshape(NB, 4 * D_IN), W1)

    htv1, dinvv = pl.pallas_call(
        _scale1_body,
        out_shape=[
            jax.ShapeDtypeStruct((HT_V, 128), f32),
            jax.ShapeDtypeStruct((HT_V, 128), f32),
        ],
    )(degp.reshape(NC, P_V, 128), h1v)

    p1 = _mp_kernel(htv1.reshape(N, D_H), edge_index, zeros_rows)

    htv2 = pl.pallas_call(
        _layer2_body,
        out_shape=jax.ShapeDtypeStruct((HT_V, 128), f32),
    )(p1.reshape(NC, P_V, 128), htv1, dinvv, b1.reshape(1, D_H), W2)

    p2 = _mp_kernel(htv2.reshape(N, D_H), edge_index, zeros_rows)

    out = pl.pallas_call(
        _final_body,
        out_shape=jax.ShapeDtypeStruct((G, 2), f32),
    )(p2.reshape(NC, P_V, 128), htv2, dinvv, b2.reshape(1, D_H),
      batch.reshape(NB, 4).T, Wc, bc.reshape(1, 2))
    return out


# speedup vs baseline: 45.0263x; 1.1607x over previous
import functools

import jax
import jax.numpy as jnp
from jax import lax
from jax.experimental import pallas as pl
from jax.experimental.pallas import tpu as pltpu
from jax.experimental.pallas import tpu_sc as plsc

N = 10000
E = 160000
D_IN = 256
D_H = 32
G = 64

NC = 2
NS = 16
NW = NC * NS
CHUNK = 200
CPW = 25
EW = CHUNK * CPW
RPT = 640
ACC_ROWS = RPT * NS
ROW_BLK = 2000
N_BLKS = N // ROW_BLK

_mesh = plsc.VectorSubcoreMesh(
    core_axis_name="c", subcore_axis_name="s", num_cores=NC, num_subcores=NS)

_sc_params = pltpu.CompilerParams(use_tc_tiling_on_sc=False)


@functools.partial(
    pl.kernel,
    out_type=jax.ShapeDtypeStruct((NC, ACC_ROWS, D_H), jnp.float32),
    mesh=_mesh,
    compiler_params=_sc_params,
    scratch_types=[
        pltpu.VMEM_SHARED((ACC_ROWS, D_H), jnp.float32),
        pltpu.VMEM((RPT // 2, D_H), jnp.float32),
        pltpu.VMEM((EW,), jnp.int32),
        pltpu.VMEM((CHUNK, D_H), jnp.float32),
    ],
)
def _deg_kernel(edge_index, zeros_hbm, ones_hbm, out, acc, stage, dstv,
                onesv):
    cid = lax.axis_index("c")
    sid = lax.axis_index("s")
    wid = cid * NS + sid
    half = RPT // 2
    pltpu.sync_copy(zeros_hbm, stage)
    for hf in range(2):
        pltpu.sync_copy(stage, acc.at[pl.ds(sid * RPT + hf * half, half)])
    pltpu.sync_copy(ones_hbm, onesv)
    pltpu.sync_copy(edge_index.at[1].at[pl.ds(wid * EW, EW)], dstv)
    plsc.subcore_barrier()

    def body(j, _):
        pltpu.sync_copy(onesv, acc.at[dstv.at[pl.ds(j * CHUNK, CHUNK)]],
                        add=True)
        return 0

    lax.fori_loop(0, CPW, body, 0)
    plsc.subcore_barrier()
    for hf in range(2):
        hr = pl.ds(sid * RPT + hf * half, half)
        pltpu.sync_copy(acc.at[hr], stage)
        pltpu.sync_copy(stage, out.at[cid].at[hr])


@functools.partial(
    pl.kernel,
    out_type=jax.ShapeDtypeStruct((NC, ACC_ROWS, D_H), jnp.float32),
    mesh=_mesh,
    compiler_params=_sc_params,
    scratch_types=[
        pltpu.VMEM_SHARED((ACC_ROWS, D_H), jnp.float32),
        pltpu.VMEM((RPT // 2, D_H), jnp.float32),
        pltpu.VMEM((EW,), jnp.int32),
        pltpu.VMEM((EW,), jnp.int32),
        pltpu.VMEM((CHUNK, D_H), jnp.float32),
        pltpu.VMEM((CHUNK, D_H), jnp.float32),
        pltpu.VMEM((CHUNK, D_H), jnp.float32),
        pltpu.SemaphoreType.DMA,
        pltpu.SemaphoreType.DMA,
        pltpu.SemaphoreType.DMA,
    ],
)
def _mp_kernel(ht, edge_index, zeros_hbm, out, acc, stage, srcv,
               dstv, rows0, rows1, rows2, sem0, sem1, sem2):
    cid = lax.axis_index("c")
    sid = lax.axis_index("s")
    wid = cid * NS + sid
    half = RPT // 2
    pltpu.sync_copy(zeros_hbm, stage)
    for hf in range(2):
        hr = pl.ds(sid * RPT + hf * half, half)
        pltpu.sync_copy(stage, acc.at[hr])
    base = wid * EW
    pltpu.sync_copy(edge_index.at[0].at[pl.ds(base, EW)], srcv)
    pltpu.sync_copy(edge_index.at[1].at[pl.ds(base, EW)], dstv)
    plsc.subcore_barrier()

    rows = (rows0, rows1, rows2)
    sems = (sem0, sem1, sem2)
    pltpu.async_copy(ht.at[srcv.at[pl.ds(0, CHUNK)]], rows0, sem0)
    pltpu.async_copy(ht.at[srcv.at[pl.ds(CHUNK, CHUNK)]], rows1, sem1)

    def body(j3, _):
        for p in range(3):
            j = j3 * 3 + p
            pltpu.make_async_copy(ht.at[srcv.at[pl.ds(0, CHUNK)]],
                                  rows[p], sems[p]).wait()

            @pl.when(j + 2 < CPW)
            def _():
                nxt = pl.ds((j + 2) * CHUNK, CHUNK)
                q = (p + 2) % 3
                pltpu.async_copy(ht.at[srcv.at[nxt]], rows[q], sems[q])

            cur = pl.ds(j * CHUNK, CHUNK)
            pltpu.sync_copy(rows[p], acc.at[dstv.at[cur]], add=True)
        return 0

    lax.fori_loop(0, (CPW - 1) // 3, body, 0)
    pltpu.make_async_copy(ht.at[srcv.at[pl.ds(0, CHUNK)]], rows0,
                          sem0).wait()
    pltpu.sync_copy(rows0, acc.at[dstv.at[pl.ds((CPW - 1) * CHUNK, CHUNK)]],
                    add=True)
    plsc.subcore_barrier()
    for hf in range(2):
        hr = pl.ds(sid * RPT + hf * half, half)
        pltpu.sync_copy(acc.at[hr], stage)
        pltpu.sync_copy(stage, out.at[cid].at[hr])


P_V = ACC_ROWS * D_H // 128
HT_V = N * D_H // 128
NB = N // 4


def _blockdiag4(w, d):
    cols = []
    for k in range(4):
        parts = []
        if k > 0:
            parts.append(jnp.zeros((k * d, D_H), jnp.float32))
        parts.append(w)
        if k < 3:
            parts.append(jnp.zeros(((3 - k) * d, D_H), jnp.float32))
        cols.append(jnp.concatenate(parts, axis=0))
    return jnp.concatenate(cols, axis=1)


def _tile4(v):
    return jnp.concatenate([v, v, v, v], axis=1)


def _layer1_body(x4_ref, w1_ref, h1_ref):
    w1b = _blockdiag4(w1_ref[...], D_IN)
    h1_ref[...] = jnp.dot(x4_ref[...], w1b,
                          preferred_element_type=jnp.float32)


def _scale1_body(degp_ref, h1_ref, ht_ref, dinv_ref):
    deg = degp_ref[0] + degp_ref[1] + 1.0
    dinv = lax.rsqrt(deg)[:HT_V]
    dinv_ref[...] = dinv
    ht_ref[...] = h1_ref[...] * dinv


def _layer2_body(p_ref, ht1_ref, dinv_ref, b1_ref, w2_ref, ht2_ref):
    p = p_ref[0][:HT_V] + p_ref[1][:HT_V]
    dinv = dinv_ref[...]
    out1 = dinv * (p + ht1_ref[...]) + _tile4(b1_ref[...])
    r = jnp.maximum(out1, 0.0)
    w2b = _blockdiag4(w2_ref[...], D_H)
    h2 = jnp.dot(r, w2b, preferred_element_type=jnp.float32)
    ht2_ref[...] = h2 * dinv


def _final_body(p_ref, ht2_ref, dinv_ref, b2_ref, batcht_ref, wc_ref, bc_ref,
                out_ref):
    p = p_ref[0][:HT_V] + p_ref[1][:HT_V]
    out2 = dinv_ref[...] * (p + ht2_ref[...]) + _tile4(b2_ref[...])
    pooled = jnp.zeros((G, D_H), jnp.float32)
    counts = jnp.zeros((G, 1), jnp.float32)
    gids = lax.broadcasted_iota(jnp.int32, (G, NB), 0)
    for k in range(4):
        bk = batcht_ref[k:k + 1, :]
        ohk = (gids == jnp.broadcast_to(bk, (G, NB))).astype(jnp.float32)
        pk = jnp.dot(ohk, out2, preferred_element_type=jnp.float32)
        pooled = pooled + pk[:, k * D_H:(k + 1) * D_H]
        counts = counts + jnp.sum(ohk, axis=1, keepdims=True)
    pooled = pooled / jnp.maximum(counts, 1.0)
    logits = (jnp.dot(pooled, wc_ref[...], preferred_element_type=jnp.float32)
              + bc_ref[...])
    m = jnp.max(logits, axis=1, keepdims=True)
    lse = jnp.log(jnp.sum(jnp.exp(logits - m), axis=1, keepdims=True)) + m
    out_ref[...] = logits - lse


def kernel(x, edge_index, batch, W1, b1, W2, b2, Wc, bc):
    f32 = jnp.float32
    zeros_rows = jnp.zeros((RPT // 2, D_H), f32)
    ones_rows = jnp.ones((CHUNK, D_H), f32)

    h1v = pl.pallas_call(
        _layer1_body,
        out_shape=jax.ShapeDtypeStruct((HT_V, 128), f32),
    )(x.reshape(NB, 4 * D_IN), W1)

    degp = _deg_kernel(edge_index, zeros_rows, ones_rows)

    htv1, dinvv = pl.pallas_call(
        _scale1_body,
        out_shape=[
            jax.ShapeDtypeStruct((HT_V, 128), f32),
            jax.ShapeDtypeStruct((HT_V, 128), f32),
        ],
    )(degp.reshape(NC, P_V, 128), h1v)

    p1 = _mp_kernel(htv1.reshape(N, D_H), edge_index, zeros_rows)

    htv2 = pl.pallas_call(
        _layer2_body,
        out_shape=jax.ShapeDtypeStruct((HT_V, 128), f32),
    )(p1.reshape(NC, P_V, 128), htv1, dinvv, b1.reshape(1, D_H), W2)

    p2 = _mp_kernel(htv2.reshape(N, D_H), edge_index, zeros_rows)

    out = pl.pallas_call(
        _final_body,
        out_shape=jax.ShapeDtypeStruct((G, 2), f32),
    )(p2.reshape(NC, P_V, 128), htv2, dinvv, b2.reshape(1, D_H),
      batch.reshape(NB, 4).T, Wc, bc.reshape(1, 2))
    return out
